# baseline (speedup 1.0000x reference)
import jax
import jax.numpy as jnp
from jax import lax
from jax.experimental import pallas as pl
from jax.experimental.pallas import tpu as pltpu
from jax.experimental.pallas import tpu_sc as plsc

D_MODEL = 128
BATCH = 16384
LANES = 16

_info = plsc.get_sparse_core_info()
NUM_CORES = _info.num_cores
NUM_SUBCORES = _info.num_subcores
NW = NUM_CORES * NUM_SUBCORES
BPW = BATCH // NW
CHUNK = 64
NCHUNK = BPW // CHUNK
DEPTH = 5


def _body(x_hbm, t_hbm, pe_hbm, out_hbm, idx_v, x_big, *rest):
    pe_bufs = rest[:DEPTH]
    gsems = rest[DEPTH:2 * DEPTH]
    xsems = rest[2 * DEPTH:2 * DEPTH + NCHUNK]
    osem = rest[2 * DEPTH + NCHUNK]

    wid = lax.axis_index("s") * NUM_CORES + lax.axis_index("c")
    base = wid * BPW
    pltpu.sync_copy(t_hbm.at[pl.ds(base, BPW)], idx_v)

    copies = [None] * NCHUNK
    xcopies = [None] * NCHUNK
    for ci in range(NCHUNK):
        if ci < DEPTH:
            copies[ci] = pltpu.async_copy(
                pe_hbm.at[idx_v.at[pl.ds(ci * CHUNK, CHUNK)]],
                pe_bufs[ci], gsems[ci])
        xcopies[ci] = pltpu.async_copy(
            x_hbm.at[pl.ds(base + ci * CHUNK, CHUNK)],
            x_big.at[pl.ds(ci * CHUNK, CHUNK)], xsems[ci])

    stores = []
    for ci in range(NCHUNK):
        k = ci % DEPTH
        copies[ci].wait()
        xcopies[ci].wait()
        pe_b = pe_bufs[k]

        @plsc.parallel_loop(0, CHUNK, unroll=2)
        def _row(r):
            xr = ci * CHUNK + r
            for j in range(D_MODEL // LANES):
                sl = pl.ds(j * LANES, LANES)
                plsc.addupdate(x_big.at[xr, sl], pe_b[r, sl])

        if ci + DEPTH < NCHUNK:
            copies[ci + DEPTH] = pltpu.async_copy(
                pe_hbm.at[idx_v.at[pl.ds((ci + DEPTH) * CHUNK, CHUNK)]],
                pe_b, gsems[k])
        stores.append(pltpu.async_copy(
            x_big.at[pl.ds(ci * CHUNK, CHUNK)],
            out_hbm.at[pl.ds(base + ci * CHUNK, CHUNK)], osem))
    for s in stores:
        s.wait()


@jax.jit
def _run(x, t, pe2d):
    mesh = plsc.VectorSubcoreMesh(core_axis_name="c", subcore_axis_name="s")
    k = pl.kernel(
        _body,
        mesh=mesh,
        out_type=jax.ShapeDtypeStruct((BATCH, D_MODEL), jnp.float32),
        scratch_types=(
            [pltpu.VMEM((BPW,), jnp.int32),
             pltpu.VMEM((BPW, D_MODEL), jnp.float32)]
            + [pltpu.VMEM((CHUNK, D_MODEL), jnp.float32)] * DEPTH
            + [pltpu.SemaphoreType.DMA] * (DEPTH + NCHUNK + 1)
        ),
    )
    return k(x, t, pe2d)


def kernel(x, t, pe):
    out = _run(x, t.astype(jnp.int32), pe.reshape(pe.shape[1], pe.shape[2]))
    return out[None]


# speedup vs baseline: 1.0265x; 1.0265x over previous
import jax
import jax.numpy as jnp
from jax import lax
from jax.experimental import pallas as pl
from jax.experimental.pallas import tpu as pltpu
from jax.experimental.pallas import tpu_sc as plsc

D_MODEL = 128
BATCH = 16384
LANES = 16

_info = plsc.get_sparse_core_info()
NUM_CORES = _info.num_cores
NUM_SUBCORES = _info.num_subcores
NW = NUM_CORES * NUM_SUBCORES
BPW = BATCH // NW
CHUNK = 128
NCHUNK = BPW // CHUNK
DEPTH = 3


def _body(x_hbm, t_hbm, pe_hbm, out_hbm, idx_v, x_big, *rest):
    pe_bufs = rest[:DEPTH]
    gsems = rest[DEPTH:2 * DEPTH]
    xsems = rest[2 * DEPTH:2 * DEPTH + NCHUNK]
    osem = rest[2 * DEPTH + NCHUNK]

    wid = lax.axis_index("s") * NUM_CORES + lax.axis_index("c")
    base = wid * BPW
    pltpu.sync_copy(t_hbm.at[pl.ds(base, BPW)], idx_v)

    copies = [None] * NCHUNK
    xcopies = [None] * NCHUNK
    for ci in range(NCHUNK):
        if ci < DEPTH:
            copies[ci] = pltpu.async_copy(
                pe_hbm.at[idx_v.at[pl.ds(ci * CHUNK, CHUNK)]],
                pe_bufs[ci], gsems[ci])
        xcopies[ci] = pltpu.async_copy(
            x_hbm.at[pl.ds(base + ci * CHUNK, CHUNK)],
            x_big.at[pl.ds(ci * CHUNK, CHUNK)], xsems[ci])

    stores = []
    for ci in range(NCHUNK):
        k = ci % DEPTH
        with jax.named_scope(f"wait{ci}"):
            copies[ci].wait()
            xcopies[ci].wait()
        pe_b = pe_bufs[k]

        with jax.named_scope(f"add{ci}"):
            @plsc.parallel_loop(0, CHUNK, unroll=2)
            def _row(r):
                xr = ci * CHUNK + r
                for j in range(D_MODEL // LANES):
                    sl = pl.ds(j * LANES, LANES)
                    plsc.addupdate(x_big.at[xr, sl], pe_b[r, sl])

        if ci + DEPTH < NCHUNK:
            copies[ci + DEPTH] = pltpu.async_copy(
                pe_hbm.at[idx_v.at[pl.ds((ci + DEPTH) * CHUNK, CHUNK)]],
                pe_b, gsems[k])
        stores.append(pltpu.async_copy(
            x_big.at[pl.ds(ci * CHUNK, CHUNK)],
            out_hbm.at[pl.ds(base + ci * CHUNK, CHUNK)], osem))
    with jax.named_scope("drain"):
        for s in stores:
            s.wait()


@jax.jit
def _run(x, t, pe2d):
    mesh = plsc.VectorSubcoreMesh(core_axis_name="c", subcore_axis_name="s")
    k = pl.kernel(
        _body,
        mesh=mesh,
        out_type=jax.ShapeDtypeStruct((BATCH, D_MODEL), jnp.float32),
        scratch_types=(
            [pltpu.VMEM((BPW,), jnp.int32),
             pltpu.VMEM((BPW, D_MODEL), jnp.float32)]
            + [pltpu.VMEM((CHUNK, D_MODEL), jnp.float32)] * DEPTH
            + [pltpu.SemaphoreType.DMA] * (DEPTH + NCHUNK + 1)
        ),
    )
    return k(x, t, pe2d)


def kernel(x, t, pe):
    out = _run(x, t.astype(jnp.int32), pe.reshape(pe.shape[1], pe.shape[2]))
    return out[None]
